# baseline (speedup 1.0000x reference)
import jax
import jax.numpy as jnp
from jax import lax
from jax.experimental import pallas as pl
from jax.experimental.pallas import tpu as pltpu
from jax.experimental.pallas import tpu_sc as plsc

N_CORES = 2
N_SUBCORES = 16
N_WORKERS = N_CORES * N_SUBCORES
CHUNK = 64
ROWS_PER_TILE = 632
NP = N_SUBCORES * ROWS_PER_TILE
DEG_W = 128
BR = 2000


def _cdiv(a, b):
    return (a + b - 1) // b


T_CORE0 = 118
T_CORE1 = 200


def _pack_edges(vec, fill, e):
    c0, c1 = T_CORE0 * CHUNK, T_CORE1 * CHUNK
    cm = max(c0, c1)
    total = N_SUBCORES * (c0 + c1)
    assert total >= e
    v = jnp.concatenate([vec, jnp.full((total - e,), fill, jnp.int32)])
    p0 = v[:N_SUBCORES * c0].reshape(N_SUBCORES, c0)
    p1 = v[N_SUBCORES * c0:].reshape(N_SUBCORES, c1)
    p0 = jnp.pad(p0, ((0, 0), (0, cm - c0)), constant_values=fill)
    p1 = jnp.pad(p1, ((0, 0), (0, cm - c1)), constant_values=fill)
    return jnp.concatenate([p0, p1], axis=0)


def _sc_mesh():
    return plsc.VectorSubcoreMesh(
        core_axis_name="c", subcore_axis_name="s",
        num_cores=N_CORES, num_subcores=N_SUBCORES)


def _make_agg(dout):
    TM = max(T_CORE0, T_CORE1)
    assert T_CORE0 % 2 == 0 and T_CORE1 % 2 == 0
    assert min(T_CORE0, T_CORE1) >= 4

    def body(src_hbm, dst_hbm, hn_hbm, zacc_hbm, acc_out,
             src_v, dst_v, rows0, rows1, acc_sh, gs0, gs1, ss0, ss1):
        cid = lax.axis_index("c")
        sid = lax.axis_index("s")
        wid = cid * N_SUBCORES + sid
        T = lax.select(cid == 0, T_CORE0, T_CORE1)
        r0 = sid * ROWS_PER_TILE
        pltpu.sync_copy(zacc_hbm.at[pl.ds(r0, ROWS_PER_TILE)],
                        acc_sh.at[pl.ds(r0, ROWS_PER_TILE)])
        pltpu.sync_copy(src_hbm.at[wid], src_v)
        pltpu.sync_copy(dst_hbm.at[wid], dst_v)
        plsc.subcore_barrier()

        rows = (rows0, rows1)
        gsem = (gs0, gs1)
        ssem = (ss0, ss1)

        def g(c, b):
            base = pl.multiple_of(c * CHUNK, CHUNK)
            pltpu.async_copy(hn_hbm.at[src_v.at[pl.ds(base, CHUNK)]],
                             rows[b], gsem[b])

        def gwait(b):
            pltpu.make_async_copy(hn_hbm.at[pl.ds(0, CHUNK)], rows[b],
                                  gsem[b]).wait()

        def s(c, b):
            base = pl.multiple_of(c * CHUNK, CHUNK)
            pltpu.async_copy(rows[b], acc_sh.at[dst_v.at[pl.ds(base, CHUNK)]],
                             ssem[b], add=True)

        def swait(b):
            pltpu.make_async_copy(rows[b], acc_sh.at[pl.ds(0, CHUNK)],
                                  ssem[b]).wait()

        g(0, 0)
        gwait(0)
        s(0, 0)
        g(1, 1)

        def step(t, carry):
            c = 2 * t
            gwait(1)
            s(c + 1, 1)
            swait(0)
            g(c + 2, 0)
            gwait(0)
            s(c + 2, 0)
            swait(1)
            g(c + 3, 1)
            return carry

        lax.fori_loop(0, T // 2 - 1, step, 0)
        gwait(1)
        s(T - 1, 1)
        swait(0)
        swait(1)
        plsc.subcore_barrier()
        pltpu.sync_copy(acc_sh.at[pl.ds(r0, ROWS_PER_TILE)],
                        acc_out.at[cid, pl.ds(r0, ROWS_PER_TILE)])

    return pl.kernel(
        body,
        out_type=jax.ShapeDtypeStruct((N_CORES, NP, dout), jnp.float32),
        mesh=_sc_mesh(),
        scratch_types=[
            pltpu.VMEM((TM * CHUNK,), jnp.int32),
            pltpu.VMEM((TM * CHUNK,), jnp.int32),
            pltpu.VMEM((CHUNK, dout), jnp.float32),
            pltpu.VMEM((CHUNK, dout), jnp.float32),
            pltpu.VMEM_SHARED((NP, dout), jnp.float32),
            pltpu.SemaphoreType.DMA,
            pltpu.SemaphoreType.DMA,
            pltpu.SemaphoreType.DMA,
            pltpu.SemaphoreType.DMA,
        ])


def _make_deg():
    TM = max(T_CORE0, T_CORE1)

    def body(dst_hbm, zdeg_hbm, ones_hbm,
             deg_out, dst_v, ones_v, deg_sh):
        cid = lax.axis_index("c")
        sid = lax.axis_index("s")
        wid = cid * N_SUBCORES + sid
        T = lax.select(cid == 0, T_CORE0, T_CORE1)
        r0 = sid * ROWS_PER_TILE
        pltpu.sync_copy(zdeg_hbm.at[pl.ds(r0, ROWS_PER_TILE)],
                        deg_sh.at[pl.ds(r0, ROWS_PER_TILE)])
        pltpu.sync_copy(ones_hbm, ones_v)
        pltpu.sync_copy(dst_hbm.at[wid], dst_v)
        plsc.subcore_barrier()

        def step(j, carry):
            base = pl.multiple_of(j * CHUNK, CHUNK)
            pltpu.sync_copy(ones_v, deg_sh.at[dst_v.at[pl.ds(base, CHUNK)]],
                            add=True)
            return carry

        lax.fori_loop(0, T, step, 0)
        plsc.subcore_barrier()
        pltpu.sync_copy(deg_sh.at[pl.ds(r0, ROWS_PER_TILE)],
                        deg_out.at[cid, pl.ds(r0, ROWS_PER_TILE)])

    return pl.kernel(
        body,
        out_type=jax.ShapeDtypeStruct((N_CORES, NP, DEG_W), jnp.float32),
        mesh=_sc_mesh(),
        scratch_types=[
            pltpu.VMEM((TM * CHUNK,), jnp.int32),
            pltpu.VMEM((CHUNK, DEG_W), jnp.float32),
            pltpu.VMEM_SHARED((NP, DEG_W), jnp.float32),
        ])


def _tc_pre(x, Wself, Wneigh, b):
    n, din = x.shape
    dout = Wself.shape[1]
    grid = n // BR

    def body(x_ref, ws_ref, wn_ref, b_ref, hs_ref, hn_ref):
        xb = x_ref[...]
        hs_ref[...] = (jnp.dot(xb, ws_ref[...], preferred_element_type=jnp.float32)
                       + b_ref[...])
        hn_ref[...] = jnp.dot(xb, wn_ref[...], preferred_element_type=jnp.float32)

    return pl.pallas_call(
        body,
        grid=(grid,),
        in_specs=[
            pl.BlockSpec((BR, din), lambda i: (i, 0)),
            pl.BlockSpec((din, dout), lambda i: (0, 0)),
            pl.BlockSpec((din, dout), lambda i: (0, 0)),
            pl.BlockSpec((1, dout), lambda i: (0, 0)),
        ],
        out_specs=[
            pl.BlockSpec((BR, dout), lambda i: (i, 0)),
            pl.BlockSpec((BR, dout), lambda i: (i, 0)),
        ],
        out_shape=[
            jax.ShapeDtypeStruct((n, dout), jnp.float32),
            jax.ShapeDtypeStruct((n, dout), jnp.float32),
        ],
    )(x, Wself, Wneigh, b.reshape(1, dout))


def _tc_mid(hsp, accA, accB, degA, degB, Wself, Wneigh, b):
    n, din = hsp.shape
    dout_s = Wself.shape[1]
    dout_n = Wneigh.shape[1]
    grid = n // BR

    def body(hsp_ref, aA_ref, aB_ref, dA_ref, dB_ref, ws_ref, wn_ref, b_ref,
             hs_ref, hn_ref):
        deg = dA_ref[:, :1] + dB_ref[:, :1]
        mean = (aA_ref[...] + aB_ref[...]) / jnp.maximum(deg, 1.0)
        h = jnp.maximum(hsp_ref[...] + mean, 0.0)
        hs_ref[...] = (jnp.dot(h, ws_ref[...], preferred_element_type=jnp.float32)
                       + b_ref[...])
        hn_ref[...] = jnp.dot(h, wn_ref[...], preferred_element_type=jnp.float32)

    return pl.pallas_call(
        body,
        grid=(grid,),
        in_specs=[
            pl.BlockSpec((BR, din), lambda i: (i, 0)),
            pl.BlockSpec((BR, din), lambda i: (i, 0)),
            pl.BlockSpec((BR, din), lambda i: (i, 0)),
            pl.BlockSpec((BR, DEG_W), lambda i: (i, 0)),
            pl.BlockSpec((BR, DEG_W), lambda i: (i, 0)),
            pl.BlockSpec((din, dout_s), lambda i: (0, 0)),
            pl.BlockSpec((din, dout_n), lambda i: (0, 0)),
            pl.BlockSpec((1, dout_s), lambda i: (0, 0)),
        ],
        out_specs=[
            pl.BlockSpec((BR, dout_s), lambda i: (i, 0)),
            pl.BlockSpec((BR, dout_n), lambda i: (i, 0)),
        ],
        out_shape=[
            jax.ShapeDtypeStruct((n, dout_s), jnp.float32),
            jax.ShapeDtypeStruct((n, dout_n), jnp.float32),
        ],
    )(hsp, accA, accB, degA, degB, Wself, Wneigh, b.reshape(1, dout_s))


def _tc_post(hsp, accA, accB, degA, degB):
    n, dout = hsp.shape
    dacc = accA.shape[1]
    grid = n // BR

    def body(hsp_ref, aA_ref, aB_ref, dA_ref, dB_ref, out_ref):
        deg = dA_ref[:, :1] + dB_ref[:, :1]
        mean = (aA_ref[:, :dout] + aB_ref[:, :dout]) / jnp.maximum(deg, 1.0)
        out_ref[...] = hsp_ref[...] + mean

    return pl.pallas_call(
        body,
        grid=(grid,),
        in_specs=[
            pl.BlockSpec((BR, dout), lambda i: (i, 0)),
            pl.BlockSpec((BR, dacc), lambda i: (i, 0)),
            pl.BlockSpec((BR, dacc), lambda i: (i, 0)),
            pl.BlockSpec((BR, DEG_W), lambda i: (i, 0)),
            pl.BlockSpec((BR, DEG_W), lambda i: (i, 0)),
        ],
        out_specs=pl.BlockSpec((BR, dout), lambda i: (i, 0)),
        out_shape=jax.ShapeDtypeStruct((n, dout), jnp.float32),
    )(hsp, accA, accB, degA, degB)


def kernel(x, edge_index, Wself0, Wneigh0, b0, Wself1, Wneigh1, b1,
           Wself2, Wneigh2, b2):
    n = x.shape[0]
    e = edge_index.shape[1]

    src_p = _pack_edges(edge_index[0].astype(jnp.int32), 0, e)
    dst_p = _pack_edges(edge_index[1].astype(jnp.int32), n, e)

    z128 = jnp.zeros((NP, 128), jnp.float32)
    zdeg = jnp.zeros((NP, DEG_W), jnp.float32)
    ones = jnp.ones((CHUNK, DEG_W), jnp.float32)

    Wneigh2p = jnp.concatenate(
        [Wneigh2, jnp.zeros((Wneigh2.shape[0], 128 - Wneigh2.shape[1]),
                            jnp.float32)], axis=1)

    agg128 = _make_agg(128)
    deg_k = _make_deg()

    deg = deg_k(dst_p, zdeg, ones)
    hs0, hn0 = _tc_pre(x, Wself0, Wneigh0, b0)
    acc0 = agg128(src_p, dst_p, hn0, z128)
    hs1, hn1 = _tc_mid(hs0, acc0[0], acc0[1], deg[0], deg[1],
                       Wself1, Wneigh1, b1)
    acc1 = agg128(src_p, dst_p, hn1, z128)
    hs2, hn2 = _tc_mid(hs1, acc1[0], acc1[1], deg[0], deg[1],
                       Wself2, Wneigh2p, b2)
    acc2 = agg128(src_p, dst_p, hn2, z128)
    out = _tc_post(hs2, acc2[0], acc2[1], deg[0], deg[1])
    return out


# speedup vs baseline: 1.5564x; 1.5564x over previous
import jax
import jax.numpy as jnp
from jax import lax
from jax.experimental import pallas as pl
from jax.experimental.pallas import tpu as pltpu
from jax.experimental.pallas import tpu_sc as plsc

N_CORES = 2
N_SUBCORES = 16
N_WORKERS = N_CORES * N_SUBCORES
CHUNK = 64
ROWS_PER_TILE = 632
NP = N_SUBCORES * ROWS_PER_TILE
DEG_W = 128
BR = 2000


def _cdiv(a, b):
    return (a + b - 1) // b


T_CORE0 = 158
T_CORE1 = 158


def _pack_edges(vec, fill, e):
    c0, c1 = T_CORE0 * CHUNK, T_CORE1 * CHUNK
    cm = max(c0, c1)
    total = N_SUBCORES * (c0 + c1)
    assert total >= e
    v = jnp.concatenate([vec, jnp.full((total - e,), fill, jnp.int32)])
    p0 = v[:N_SUBCORES * c0].reshape(N_SUBCORES, c0)
    p1 = v[N_SUBCORES * c0:].reshape(N_SUBCORES, c1)
    p0 = jnp.pad(p0, ((0, 0), (0, cm - c0)), constant_values=fill)
    p1 = jnp.pad(p1, ((0, 0), (0, cm - c1)), constant_values=fill)
    return jnp.concatenate([p0, p1], axis=0)


def _sc_mesh():
    return plsc.VectorSubcoreMesh(
        core_axis_name="c", subcore_axis_name="s",
        num_cores=N_CORES, num_subcores=N_SUBCORES)


def _make_agg(dout):
    TM = max(T_CORE0, T_CORE1)
    assert T_CORE0 % 2 == 0 and T_CORE1 % 2 == 0
    assert min(T_CORE0, T_CORE1) >= 4

    def body(src_hbm, dst_hbm, hn_hbm, zacc_hbm, acc_out,
             src_v, dst_v, rows0, rows1, acc_sh, gs0, gs1, ss0, ss1):
        cid = lax.axis_index("c")
        sid = lax.axis_index("s")
        wid = cid * N_SUBCORES + sid
        T = lax.select(cid == 0, T_CORE0, T_CORE1)
        r0 = sid * ROWS_PER_TILE
        pltpu.sync_copy(zacc_hbm.at[pl.ds(r0, ROWS_PER_TILE)],
                        acc_sh.at[pl.ds(r0, ROWS_PER_TILE)])
        pltpu.sync_copy(src_hbm.at[wid], src_v)
        pltpu.sync_copy(dst_hbm.at[wid], dst_v)
        plsc.subcore_barrier()

        rows = (rows0, rows1)
        gsem = (gs0, gs1)
        ssem = (ss0, ss1)

        def g(c, b):
            base = pl.multiple_of(c * CHUNK, CHUNK)
            pltpu.async_copy(hn_hbm.at[src_v.at[pl.ds(base, CHUNK)]],
                             rows[b], gsem[b])

        def gwait(b):
            pltpu.make_async_copy(hn_hbm.at[pl.ds(0, CHUNK)], rows[b],
                                  gsem[b]).wait()

        def s(c, b):
            base = pl.multiple_of(c * CHUNK, CHUNK)
            pltpu.async_copy(rows[b], acc_sh.at[dst_v.at[pl.ds(base, CHUNK)]],
                             ssem[b], add=True)

        def swait(b):
            pltpu.make_async_copy(rows[b], acc_sh.at[pl.ds(0, CHUNK)],
                                  ssem[b]).wait()

        g(0, 0)
        gwait(0)
        s(0, 0)
        g(1, 1)

        def step(t, carry):
            c = 2 * t
            gwait(1)
            s(c + 1, 1)
            swait(0)
            g(c + 2, 0)
            gwait(0)
            s(c + 2, 0)
            swait(1)
            g(c + 3, 1)
            return carry

        lax.fori_loop(0, T // 2 - 1, step, 0)
        gwait(1)
        s(T - 1, 1)
        swait(0)
        swait(1)
        plsc.subcore_barrier()
        pltpu.sync_copy(acc_sh.at[pl.ds(r0, ROWS_PER_TILE)],
                        acc_out.at[cid, pl.ds(r0, ROWS_PER_TILE)])

    return pl.kernel(
        body,
        out_type=jax.ShapeDtypeStruct((N_CORES, NP, dout), jnp.float32),
        mesh=_sc_mesh(),
        scratch_types=[
            pltpu.VMEM((TM * CHUNK,), jnp.int32),
            pltpu.VMEM((TM * CHUNK,), jnp.int32),
            pltpu.VMEM((CHUNK, dout), jnp.float32),
            pltpu.VMEM((CHUNK, dout), jnp.float32),
            pltpu.VMEM_SHARED((NP, dout), jnp.float32),
            pltpu.SemaphoreType.DMA,
            pltpu.SemaphoreType.DMA,
            pltpu.SemaphoreType.DMA,
            pltpu.SemaphoreType.DMA,
        ])


def _make_deg():
    TM = max(T_CORE0, T_CORE1)

    def body(dst_hbm, zdeg_hbm, ones_hbm,
             deg_out, dst_v, ones_v, deg_sh):
        cid = lax.axis_index("c")
        sid = lax.axis_index("s")
        wid = cid * N_SUBCORES + sid
        T = lax.select(cid == 0, T_CORE0, T_CORE1)
        r0 = sid * ROWS_PER_TILE
        pltpu.sync_copy(zdeg_hbm.at[pl.ds(r0, ROWS_PER_TILE)],
                        deg_sh.at[pl.ds(r0, ROWS_PER_TILE)])
        pltpu.sync_copy(ones_hbm, ones_v)
        pltpu.sync_copy(dst_hbm.at[wid], dst_v)
        plsc.subcore_barrier()

        def step(j, carry):
            base = pl.multiple_of(j * CHUNK, CHUNK)
            pltpu.sync_copy(ones_v, deg_sh.at[dst_v.at[pl.ds(base, CHUNK)]],
                            add=True)
            return carry

        lax.fori_loop(0, T, step, 0)
        plsc.subcore_barrier()
        pltpu.sync_copy(deg_sh.at[pl.ds(r0, ROWS_PER_TILE)],
                        deg_out.at[cid, pl.ds(r0, ROWS_PER_TILE)])

    return pl.kernel(
        body,
        out_type=jax.ShapeDtypeStruct((N_CORES, NP, DEG_W), jnp.float32),
        mesh=_sc_mesh(),
        scratch_types=[
            pltpu.VMEM((TM * CHUNK,), jnp.int32),
            pltpu.VMEM((CHUNK, DEG_W), jnp.float32),
            pltpu.VMEM_SHARED((NP, DEG_W), jnp.float32),
        ])


def _tc_pre(x, Wself, Wneigh, b):
    n, din = x.shape
    dout = Wself.shape[1]
    grid = n // BR

    def body(x_ref, ws_ref, wn_ref, b_ref, hs_ref, hn_ref):
        xb = x_ref[...]
        hs_ref[...] = (jnp.dot(xb, ws_ref[...], preferred_element_type=jnp.float32)
                       + b_ref[...])
        hn_ref[...] = jnp.dot(xb, wn_ref[...], preferred_element_type=jnp.float32)

    return pl.pallas_call(
        body,
        grid=(grid,),
        in_specs=[
            pl.BlockSpec((BR, din), lambda i: (i, 0)),
            pl.BlockSpec((din, dout), lambda i: (0, 0)),
            pl.BlockSpec((din, dout), lambda i: (0, 0)),
            pl.BlockSpec((1, dout), lambda i: (0, 0)),
        ],
        out_specs=[
            pl.BlockSpec((BR, dout), lambda i: (i, 0)),
            pl.BlockSpec((BR, dout), lambda i: (i, 0)),
        ],
        out_shape=[
            jax.ShapeDtypeStruct((n, dout), jnp.float32),
            jax.ShapeDtypeStruct((n, dout), jnp.float32),
        ],
    )(x, Wself, Wneigh, b.reshape(1, dout))


def _tc_mid(hsp, accA, accB, degA, degB, Wself, Wneigh, b):
    n, din = hsp.shape
    dout_s = Wself.shape[1]
    dout_n = Wneigh.shape[1]
    grid = n // BR

    def body(hsp_ref, aA_ref, aB_ref, dA_ref, dB_ref, ws_ref, wn_ref, b_ref,
             hs_ref, hn_ref):
        deg = dA_ref[:, :1] + dB_ref[:, :1]
        mean = (aA_ref[...] + aB_ref[...]) / jnp.maximum(deg, 1.0)
        h = jnp.maximum(hsp_ref[...] + mean, 0.0)
        hs_ref[...] = (jnp.dot(h, ws_ref[...], preferred_element_type=jnp.float32)
                       + b_ref[...])
        hn_ref[...] = jnp.dot(h, wn_ref[...], preferred_element_type=jnp.float32)

    return pl.pallas_call(
        body,
        grid=(grid,),
        in_specs=[
            pl.BlockSpec((BR, din), lambda i: (i, 0)),
            pl.BlockSpec((BR, din), lambda i: (i, 0)),
            pl.BlockSpec((BR, din), lambda i: (i, 0)),
            pl.BlockSpec((BR, DEG_W), lambda i: (i, 0)),
            pl.BlockSpec((BR, DEG_W), lambda i: (i, 0)),
            pl.BlockSpec((din, dout_s), lambda i: (0, 0)),
            pl.BlockSpec((din, dout_n), lambda i: (0, 0)),
            pl.BlockSpec((1, dout_s), lambda i: (0, 0)),
        ],
        out_specs=[
            pl.BlockSpec((BR, dout_s), lambda i: (i, 0)),
            pl.BlockSpec((BR, dout_n), lambda i: (i, 0)),
        ],
        out_shape=[
            jax.ShapeDtypeStruct((n, dout_s), jnp.float32),
            jax.ShapeDtypeStruct((n, dout_n), jnp.float32),
        ],
    )(hsp, accA, accB, degA, degB, Wself, Wneigh, b.reshape(1, dout_s))


def _tc_post(hsp, accA, accB, degA, degB):
    n, dout = hsp.shape
    dacc = accA.shape[1]
    grid = n // BR

    def body(hsp_ref, aA_ref, aB_ref, dA_ref, dB_ref, out_ref):
        deg = dA_ref[:, :1] + dB_ref[:, :1]
        mean = (aA_ref[:, :dout] + aB_ref[:, :dout]) / jnp.maximum(deg, 1.0)
        out_ref[...] = hsp_ref[...] + mean

    return pl.pallas_call(
        body,
        grid=(grid,),
        in_specs=[
            pl.BlockSpec((BR, dout), lambda i: (i, 0)),
            pl.BlockSpec((BR, dacc), lambda i: (i, 0)),
            pl.BlockSpec((BR, dacc), lambda i: (i, 0)),
            pl.BlockSpec((BR, DEG_W), lambda i: (i, 0)),
            pl.BlockSpec((BR, DEG_W), lambda i: (i, 0)),
        ],
        out_specs=pl.BlockSpec((BR, dout), lambda i: (i, 0)),
        out_shape=jax.ShapeDtypeStruct((n, dout), jnp.float32),
    )(hsp, accA, accB, degA, degB)


def kernel(x, edge_index, Wself0, Wneigh0, b0, Wself1, Wneigh1, b1,
           Wself2, Wneigh2, b2):
    n = x.shape[0]
    e = edge_index.shape[1]

    src_p = _pack_edges(edge_index[0].astype(jnp.int32), 0, e)
    dst_p = _pack_edges(edge_index[1].astype(jnp.int32), n, e)

    z128 = jnp.zeros((NP, 128), jnp.float32)
    zdeg = jnp.zeros((NP, DEG_W), jnp.float32)
    ones = jnp.ones((CHUNK, DEG_W), jnp.float32)

    Wneigh2p = jnp.concatenate(
        [Wneigh2, jnp.zeros((Wneigh2.shape[0], 128 - Wneigh2.shape[1]),
                            jnp.float32)], axis=1)

    agg128 = _make_agg(128)
    deg_k = _make_deg()

    deg = deg_k(dst_p, zdeg, ones)
    hs0, hn0 = _tc_pre(x, Wself0, Wneigh0, b0)
    acc0 = agg128(src_p, dst_p, hn0, z128)
    hs1, hn1 = _tc_mid(hs0, acc0[0], acc0[1], deg[0], deg[1],
                       Wself1, Wneigh1, b1)
    acc1 = agg128(src_p, dst_p, hn1, z128)
    hs2, hn2 = _tc_mid(hs1, acc1[0], acc1[1], deg[0], deg[1],
                       Wself2, Wneigh2p, b2)
    acc2 = agg128(src_p, dst_p, hn2, z128)
    out = _tc_post(hs2, acc2[0], acc2[1], deg[0], deg[1])
    return out


# speedup vs baseline: 1.7609x; 1.1314x over previous
import jax
import jax.numpy as jnp
from jax import lax
from jax.experimental import pallas as pl
from jax.experimental.pallas import tpu as pltpu
from jax.experimental.pallas import tpu_sc as plsc

N_CORES = 2
N_SUBCORES = 16
N_WORKERS = N_CORES * N_SUBCORES
CHUNK = 64
ROWS_PER_TILE = 632
NP = N_SUBCORES * ROWS_PER_TILE
BR = 2000
HIST_ROWS = 80


def _cdiv(a, b):
    return (a + b - 1) // b


T_CORE0 = 158
T_CORE1 = 158


def _pack_edges(vec, fill, e):
    c0, c1 = T_CORE0 * CHUNK, T_CORE1 * CHUNK
    cm = max(c0, c1)
    total = N_SUBCORES * (c0 + c1)
    assert total >= e
    v = jnp.concatenate([vec, jnp.full((total - e,), fill, jnp.int32)])
    p0 = v[:N_SUBCORES * c0].reshape(N_SUBCORES, c0)
    p1 = v[N_SUBCORES * c0:].reshape(N_SUBCORES, c1)
    p0 = jnp.pad(p0, ((0, 0), (0, cm - c0)), constant_values=fill)
    p1 = jnp.pad(p1, ((0, 0), (0, cm - c1)), constant_values=fill)
    return jnp.concatenate([p0, p1], axis=0)


def _sc_mesh():
    return plsc.VectorSubcoreMesh(
        core_axis_name="c", subcore_axis_name="s",
        num_cores=N_CORES, num_subcores=N_SUBCORES)


def _make_agg(dout, with_deg=False):
    TM = max(T_CORE0, T_CORE1)
    assert T_CORE0 % 2 == 0 and T_CORE1 % 2 == 0
    assert min(T_CORE0, T_CORE1) >= 4

    def body(src_hbm, dst_hbm, hn_hbm, zacc_hbm, *rest):
        if with_deg:
            (acc_out, deg_out, src_v, dst_v, rows0, rows1, acc_sh, hist_v,
             gs0, gs1, ss0, ss1) = rest
        else:
            (acc_out, src_v, dst_v, rows0, rows1, acc_sh,
             gs0, gs1, ss0, ss1) = rest
        cid = lax.axis_index("c")
        sid = lax.axis_index("s")
        wid = cid * N_SUBCORES + sid
        T = lax.select(cid == 0, T_CORE0, T_CORE1)
        r0 = sid * ROWS_PER_TILE
        pltpu.sync_copy(zacc_hbm.at[pl.ds(r0, ROWS_PER_TILE)],
                        acc_sh.at[pl.ds(r0, ROWS_PER_TILE)])
        pltpu.sync_copy(src_hbm.at[wid], src_v)
        pltpu.sync_copy(dst_hbm.at[wid], dst_v)
        if with_deg:
            pltpu.sync_copy(zacc_hbm.at[pl.ds(0, HIST_ROWS)], hist_v)
        plsc.subcore_barrier()

        rows = (rows0, rows1)
        gsem = (gs0, gs1)
        ssem = (ss0, ss1)
        ones16 = jnp.ones((16,), jnp.float32)

        def g(c, b):
            base = pl.multiple_of(c * CHUNK, CHUNK)
            pltpu.async_copy(hn_hbm.at[src_v.at[pl.ds(base, CHUNK)]],
                             rows[b], gsem[b])

        def gwait(b):
            pltpu.make_async_copy(hn_hbm.at[pl.ds(0, CHUNK)], rows[b],
                                  gsem[b]).wait()

        def s(c, b):
            base = pl.multiple_of(c * CHUNK, CHUNK)
            pltpu.async_copy(rows[b], acc_sh.at[dst_v.at[pl.ds(base, CHUNK)]],
                             ssem[b], add=True)

        def swait(b):
            pltpu.make_async_copy(rows[b], acc_sh.at[pl.ds(0, CHUNK)],
                                  ssem[b]).wait()

        def deg(c):
            if with_deg:
                for q in range(CHUNK // 16):
                    base = pl.multiple_of(c * CHUNK + q * 16, 16)
                    dv = dst_v[pl.ds(base, 16)]
                    plsc.addupdate_scatter(
                        hist_v,
                        [lax.shift_right_logical(dv, 7),
                         jnp.bitwise_and(dv, 127)],
                        ones16)

        g(0, 0)
        gwait(0)
        s(0, 0)
        deg(0)
        g(1, 1)

        def step(t, carry):
            c = 2 * t
            gwait(1)
            s(c + 1, 1)
            swait(0)
            g(c + 2, 0)
            deg(c + 1)
            gwait(0)
            s(c + 2, 0)
            swait(1)
            g(c + 3, 1)
            deg(c + 2)
            return carry

        lax.fori_loop(0, T // 2 - 1, step, 0)
        gwait(1)
        s(T - 1, 1)
        deg(T - 1)
        swait(0)
        swait(1)
        plsc.subcore_barrier()
        pltpu.sync_copy(acc_sh.at[pl.ds(r0, ROWS_PER_TILE)],
                        acc_out.at[cid, pl.ds(r0, ROWS_PER_TILE)])
        if with_deg:
            pltpu.sync_copy(hist_v, deg_out.at[wid])

    acc_t = jax.ShapeDtypeStruct((N_CORES, NP, dout), jnp.float32)
    out_type = ([acc_t,
                 jax.ShapeDtypeStruct((N_WORKERS, HIST_ROWS, 128), jnp.float32)]
                if with_deg else acc_t)
    scratch = [
        pltpu.VMEM((TM * CHUNK,), jnp.int32),
        pltpu.VMEM((TM * CHUNK,), jnp.int32),
        pltpu.VMEM((CHUNK, dout), jnp.float32),
        pltpu.VMEM((CHUNK, dout), jnp.float32),
        pltpu.VMEM_SHARED((NP, dout), jnp.float32),
    ]
    if with_deg:
        scratch.append(pltpu.VMEM((HIST_ROWS, 128), jnp.float32))
    scratch += [pltpu.SemaphoreType.DMA] * 4

    return pl.kernel(
        body, out_type=out_type, mesh=_sc_mesh(), scratch_types=scratch,
        compiler_params=pltpu.CompilerParams(
            needs_layout_passes=False) if with_deg else None)


def _tc_pre(x, Wself, Wneigh, b):
    n, din = x.shape
    dout = Wself.shape[1]
    grid = n // BR

    def body(x_ref, ws_ref, wn_ref, b_ref, hs_ref, hn_ref):
        xb = x_ref[...]
        hs_ref[...] = (jnp.dot(xb, ws_ref[...], preferred_element_type=jnp.float32)
                       + b_ref[...])
        hn_ref[...] = jnp.dot(xb, wn_ref[...], preferred_element_type=jnp.float32)

    return pl.pallas_call(
        body,
        grid=(grid,),
        in_specs=[
            pl.BlockSpec((BR, din), lambda i: (i, 0)),
            pl.BlockSpec((din, dout), lambda i: (0, 0)),
            pl.BlockSpec((din, dout), lambda i: (0, 0)),
            pl.BlockSpec((1, dout), lambda i: (0, 0)),
        ],
        out_specs=[
            pl.BlockSpec((BR, dout), lambda i: (i, 0)),
            pl.BlockSpec((BR, dout), lambda i: (i, 0)),
        ],
        out_shape=[
            jax.ShapeDtypeStruct((n, dout), jnp.float32),
            jax.ShapeDtypeStruct((n, dout), jnp.float32),
        ],
    )(x, Wself, Wneigh, b.reshape(1, dout))


def _tc_mid(hsp, accA, accB, dh, Wself, Wneigh, b):
    n, din = hsp.shape
    dout_s = Wself.shape[1]
    dout_n = Wneigh.shape[1]
    grid = n // BR

    def body(hsp_ref, aA_ref, aB_ref, dh_ref, ws_ref, wn_ref, b_ref,
             hs_ref, hn_ref):
        deg = jnp.sum(dh_ref[...], axis=1, keepdims=True)
        mean = (aA_ref[...] + aB_ref[...]) / jnp.maximum(deg, 1.0)
        h = jnp.maximum(hsp_ref[...] + mean, 0.0)
        hs_ref[...] = (jnp.dot(h, ws_ref[...], preferred_element_type=jnp.float32)
                       + b_ref[...])
        hn_ref[...] = jnp.dot(h, wn_ref[...], preferred_element_type=jnp.float32)

    return pl.pallas_call(
        body,
        grid=(grid,),
        in_specs=[
            pl.BlockSpec((BR, din), lambda i: (i, 0)),
            pl.BlockSpec((BR, din), lambda i: (i, 0)),
            pl.BlockSpec((BR, din), lambda i: (i, 0)),
            pl.BlockSpec((BR, N_WORKERS), lambda i: (i, 0)),
            pl.BlockSpec((din, dout_s), lambda i: (0, 0)),
            pl.BlockSpec((din, dout_n), lambda i: (0, 0)),
            pl.BlockSpec((1, dout_s), lambda i: (0, 0)),
        ],
        out_specs=[
            pl.BlockSpec((BR, dout_s), lambda i: (i, 0)),
            pl.BlockSpec((BR, dout_n), lambda i: (i, 0)),
        ],
        out_shape=[
            jax.ShapeDtypeStruct((n, dout_s), jnp.float32),
            jax.ShapeDtypeStruct((n, dout_n), jnp.float32),
        ],
    )(hsp, accA, accB, dh, Wself, Wneigh, b.reshape(1, dout_s))


def _tc_post(hsp, accA, accB, dh):
    n, dout = hsp.shape
    dacc = accA.shape[1]
    grid = n // BR

    def body(hsp_ref, aA_ref, aB_ref, dh_ref, out_ref):
        deg = jnp.sum(dh_ref[...], axis=1, keepdims=True)
        mean = (aA_ref[:, :dout] + aB_ref[:, :dout]) / jnp.maximum(deg, 1.0)
        out_ref[...] = hsp_ref[...] + mean

    return pl.pallas_call(
        body,
        grid=(grid,),
        in_specs=[
            pl.BlockSpec((BR, dout), lambda i: (i, 0)),
            pl.BlockSpec((BR, dacc), lambda i: (i, 0)),
            pl.BlockSpec((BR, dacc), lambda i: (i, 0)),
            pl.BlockSpec((BR, N_WORKERS), lambda i: (i, 0)),
        ],
        out_specs=pl.BlockSpec((BR, dout), lambda i: (i, 0)),
        out_shape=jax.ShapeDtypeStruct((n, dout), jnp.float32),
    )(hsp, accA, accB, dh)


def kernel(x, edge_index, Wself0, Wneigh0, b0, Wself1, Wneigh1, b1,
           Wself2, Wneigh2, b2):
    n = x.shape[0]
    e = edge_index.shape[1]

    src_p = _pack_edges(edge_index[0].astype(jnp.int32), 0, e)
    dst_p = _pack_edges(edge_index[1].astype(jnp.int32), n, e)

    z128 = jnp.zeros((NP, 128), jnp.float32)

    Wneigh2p = jnp.concatenate(
        [Wneigh2, jnp.zeros((Wneigh2.shape[0], 128 - Wneigh2.shape[1]),
                            jnp.float32)], axis=1)

    agg_deg = _make_agg(128, with_deg=True)
    agg128 = _make_agg(128)

    hs0, hn0 = _tc_pre(x, Wself0, Wneigh0, b0)
    acc0, deg_hists = agg_deg(src_p, dst_p, hn0, z128)
    dh = deg_hists.reshape(N_WORKERS, HIST_ROWS * 128).T
    hs1, hn1 = _tc_mid(hs0, acc0[0], acc0[1], dh, Wself1, Wneigh1, b1)
    acc1 = agg128(src_p, dst_p, hn1, z128)
    hs2, hn2 = _tc_mid(hs1, acc1[0], acc1[1], dh, Wself2, Wneigh2p, b2)
    acc2 = agg128(src_p, dst_p, hn2, z128)
    out = _tc_post(hs2, acc2[0], acc2[1], dh)
    return out
